# Initial kernel scaffold; baseline (speedup 1.0000x reference)
import jax
import jax.numpy as jnp
from jax.experimental import pallas as pl


def kernel(embed, edge_index, H_idx, H_segment_ids, T_idx, T_segment_ids, beta, attn_w, attn_b, W1, b1, W2, b2):
    raise NotImplementedError("write your pallas kernel here")


# speedup vs baseline: 10.0489x; 10.0489x over previous
import functools

import jax
import jax.numpy as jnp
from jax import lax
from jax.experimental import pallas as pl
from jax.experimental.pallas import tpu as pltpu
from jax.experimental.pallas import tpu_sc as plsc

N = 10000
E = 320000
D = 128
KHOPS = 3
NSETS = 16
TOT = 16384
HID = 512
NREL = 64

NC, NS = 2, 16
NW = NC * NS
EPW = E // NW
EC = 80
NCHUNK = EPW // EC
NPAD = 10240
RPT = NPAD // NS
GPW = TOT // NW
GC = 128
GCHUNK = GPW // GC

@functools.lru_cache(maxsize=1)
def _sc_kernels():
    mesh = plsc.VectorSubcoreMesh(
        core_axis_name="c", subcore_axis_name="s",
        num_cores=NC, num_subcores=NS)

    @functools.partial(
        pl.kernel,
        out_type=jax.ShapeDtypeStruct((NC * NPAD,), jnp.float32),
        mesh=mesh,
        scratch_types=[
            pltpu.VMEM((EC,), jnp.int32),
            pltpu.VMEM((EC,), jnp.float32),
            pltpu.VMEM((RPT,), jnp.float32),
            pltpu.VMEM_SHARED((NPAD,), jnp.float32),
            pltpu.SemaphoreType.DMA,
        ],
    )
    def _deg_sc(dst_hbm, out_hbm, idx_v, ones_v, zbuf_v, acc, sem):
        cid = lax.axis_index("c")
        sid = lax.axis_index("s")
        wid = cid * NS + sid
        for j in range(RPT // 16):
            zbuf_v[pl.ds(j * 16, 16)] = jnp.zeros((16,), jnp.float32)
        for j in range(EC // 16):
            ones_v[pl.ds(j * 16, 16)] = jnp.ones((16,), jnp.float32)
        pltpu.sync_copy(zbuf_v, acc.at[pl.ds(sid * RPT, RPT)])
        plsc.subcore_barrier()

        def body(i, carry):
            base = pl.multiple_of(wid * EPW + i * EC, 8)
            pltpu.sync_copy(dst_hbm.at[pl.ds(base, EC)], idx_v)
            pltpu.sync_copy(ones_v, acc.at[idx_v], add=True)
            return carry

        lax.fori_loop(0, NCHUNK, body, 0)
        plsc.subcore_barrier()
        dst0 = pl.multiple_of(cid * NPAD + sid * RPT, 8)
        pltpu.sync_copy(acc.at[pl.ds(sid * RPT, RPT)],
                        out_hbm.at[pl.ds(dst0, RPT)])

    @functools.partial(
        pl.kernel,
        out_type=jax.ShapeDtypeStruct((NC * NPAD, D), jnp.float32),
        mesh=mesh,
        scratch_types=[
            pltpu.VMEM((EC,), jnp.int32),
            pltpu.VMEM((EC,), jnp.int32),
            pltpu.VMEM((EC, D), jnp.float32),
            pltpu.VMEM((128, D), jnp.float32),
            pltpu.VMEM_SHARED((NPAD, D), jnp.float32),
            pltpu.SemaphoreType.DMA,
        ],
    )
    def _hop_sc(src_hbm, dst_hbm, y_hbm, out_hbm, sidx_v, didx_v, rows_v,
                zbuf_v, acc, sem):
        cid = lax.axis_index("c")
        sid = lax.axis_index("s")
        wid = cid * NS + sid

        def zfill(i, carry):
            for j in range(D // 16):
                zbuf_v[i, pl.ds(j * 16, 16)] = jnp.zeros((16,), jnp.float32)
            return carry

        lax.fori_loop(0, 128, zfill, 0)
        for i in range(RPT // 128):
            pltpu.sync_copy(zbuf_v, acc.at[pl.ds(sid * RPT + i * 128, 128)])
        plsc.subcore_barrier()

        def body(i, carry):
            base = pl.multiple_of(wid * EPW + i * EC, 8)
            pltpu.sync_copy(src_hbm.at[pl.ds(base, EC)], sidx_v)
            pltpu.sync_copy(dst_hbm.at[pl.ds(base, EC)], didx_v)
            pltpu.async_copy(y_hbm.at[sidx_v], rows_v, sem).wait()
            pltpu.sync_copy(rows_v, acc.at[didx_v], add=True)
            return carry

        lax.fori_loop(0, NCHUNK, body, 0)
        plsc.subcore_barrier()
        dst0 = pl.multiple_of(cid * NPAD + sid * RPT, 8)
        pltpu.sync_copy(acc.at[pl.ds(sid * RPT, RPT)],
                        out_hbm.at[pl.ds(dst0, RPT)])

    @functools.partial(
        pl.kernel,
        out_type=(jax.ShapeDtypeStruct((TOT, D), jnp.float32),
                  jax.ShapeDtypeStruct((TOT, D), jnp.float32)),
        mesh=mesh,
        scratch_types=[
            pltpu.VMEM((GC,), jnp.int32),
            pltpu.VMEM((GC, D), jnp.float32),
            pltpu.SemaphoreType.DMA,
        ],
    )
    def _pool_gather_sc(z_hbm, hidx_hbm, tidx_hbm, outh_hbm, outt_hbm,
                        gidx_v, grow_v, sem):
        cid = lax.axis_index("c")
        sid = lax.axis_index("s")
        wid = cid * NS + sid

        def body(i, carry):
            base = pl.multiple_of(wid * GPW + i * GC, 8)
            pltpu.sync_copy(hidx_hbm.at[pl.ds(base, GC)], gidx_v)
            pltpu.async_copy(z_hbm.at[gidx_v], grow_v, sem).wait()
            pltpu.sync_copy(grow_v, outh_hbm.at[pl.ds(base, GC)])
            pltpu.sync_copy(tidx_hbm.at[pl.ds(base, GC)], gidx_v)
            pltpu.async_copy(z_hbm.at[gidx_v], grow_v, sem).wait()
            pltpu.sync_copy(grow_v, outt_hbm.at[pl.ds(base, GC)])
            return carry

        lax.fori_loop(0, GCHUNK, body, 0)

    return _deg_sc, _hop_sc, _pool_gather_sc


def _pre_tc(p0_ref, p1_ref, emb_ref, b0_ref, dinv_ref, y_ref, h_ref):
    deg = p0_ref[...] + p1_ref[...] + 1.0
    dinv = lax.rsqrt(jnp.maximum(deg, 1.0))
    dinv_ref[...] = dinv
    y_ref[...] = emb_ref[...] * dinv
    h_ref[...] = emb_ref[...] * b0_ref[0, 0]


def _combine_tc(p0_ref, p1_ref, y_ref, h_ref, dinv_ref, bk_ref,
                hout_ref, yout_ref):
    dinv = dinv_ref[...]
    x = (p0_ref[...] + p1_ref[...] + y_ref[...]) * dinv
    hout_ref[...] = h_ref[...] + bk_ref[0, 0] * x
    yout_ref[...] = x * dinv


def _pool_mlp_tc(subh_ref, subt_ref, hseg_ref, tseg_ref, aw_ref, ab_ref,
                 w1_ref, b1_ref, w2_ref, b2_ref, out_ref):
    hp = lax.Precision.HIGHEST
    aw = aw_ref[...]
    ab = ab_ref[0, 0]

    def pool(sub, seg_row):
        s = lax.dot_general(aw, sub, (((0,), (1,)), ((), ())),
                            precision=hp,
                            preferred_element_type=jnp.float32) + ab
        onehot = seg_row == lax.broadcasted_iota(jnp.int32, (NSETS, TOT), 0)
        m = jnp.max(jnp.where(onehot, s, -3e38), axis=1, keepdims=True)
        m_safe = jnp.where(m > -1e37, m, 0.0)
        m_per = jnp.sum(jnp.where(onehot, m_safe, 0.0), axis=0,
                        keepdims=True)
        e = jnp.exp(s - m_per)
        denom = jnp.sum(jnp.where(onehot, e, 0.0), axis=1, keepdims=True)
        safe_denom = jnp.where(denom > 0, denom, 1.0)
        wden = jnp.sum(jnp.where(onehot, safe_denom, 0.0), axis=0,
                       keepdims=True)
        w = e / wden
        wm = jnp.where(onehot, w, 0.0)
        pooled = lax.dot_general(wm, sub, (((1,), (0,)), ((), ())),
                                 precision=hp,
                                 preferred_element_type=jnp.float32)
        cnt = jnp.sum(jnp.where(onehot, 1.0, 0.0), axis=1, keepdims=True)
        return jnp.where(cnt > 0, pooled, 0.0)

    h = pool(subh_ref[...], hseg_ref[...])
    t = pool(subt_ref[...], tseg_ref[...])
    feats = jnp.concatenate([h, t, h * t], axis=-1)
    hidden = jnp.dot(feats, w1_ref[...], precision=hp,
                     preferred_element_type=jnp.float32) + b1_ref[...]
    hidden = jnp.maximum(hidden, 0.0)
    out_ref[...] = jnp.dot(hidden, w2_ref[...], precision=hp,
                           preferred_element_type=jnp.float32) + b2_ref[...]


def _tc_call(body, out_shapes, interpret=False):
    return pl.pallas_call(body, out_shape=out_shapes, interpret=interpret)


def kernel(embed, edge_index, H_idx, H_segment_ids, T_idx, T_segment_ids,
           beta, attn_w, attn_b, W1, b1, W2, b2):
    f32 = jnp.float32
    src = edge_index[0]
    dst = edge_index[1]
    _deg_sc, _hop_sc, _pool_gather_sc = _sc_kernels()

    degp = _deg_sc(dst)
    p0 = degp[:N].reshape(N, 1)
    p1 = degp[NPAD:NPAD + N].reshape(N, 1)

    dinv, y, h = _tc_call(
        _pre_tc,
        (jax.ShapeDtypeStruct((N, 1), f32),
         jax.ShapeDtypeStruct((N, D), f32),
         jax.ShapeDtypeStruct((N, D), f32)),
    )(p0, p1, embed, beta[0].reshape(1, 1))

    for k in range(KHOPS):
        parts = _hop_sc(src, dst, y)
        h, y = _tc_call(
            _combine_tc,
            (jax.ShapeDtypeStruct((N, D), f32),
             jax.ShapeDtypeStruct((N, D), f32)),
        )(parts[:N], parts[NPAD:NPAD + N], y, h, dinv,
          beta[k + 1].reshape(1, 1))

    subh, subt = _pool_gather_sc(h, H_idx, T_idx)

    logits = _tc_call(
        _pool_mlp_tc, jax.ShapeDtypeStruct((NSETS, NREL), f32),
    )(subh, subt,
      H_segment_ids.reshape(1, TOT), T_segment_ids.reshape(1, TOT),
      attn_w, attn_b.reshape(1, 1),
      W1, b1.reshape(1, HID), W2, b2.reshape(1, NREL))
    return logits


# speedup vs baseline: 24.4280x; 2.4309x over previous
import functools

import jax
import jax.numpy as jnp
from jax import lax
from jax.experimental import pallas as pl
from jax.experimental.pallas import tpu as pltpu
from jax.experimental.pallas import tpu_sc as plsc

N = 10000
E = 320000
D = 128
KHOPS = 3
NSETS = 16
TOT = 16384
HID = 512
NREL = 64

NC, NS = 2, 16
NW = NC * NS
EPW = E // NW
EC = 80
NCHUNK = EPW // EC
NPAD = 10240
RPT = NPAD // NS
GPW = TOT // NW
GC = 128
GCHUNK = GPW // GC

@functools.lru_cache(maxsize=1)
def _sc_kernels():
    mesh = plsc.VectorSubcoreMesh(
        core_axis_name="c", subcore_axis_name="s",
        num_cores=NC, num_subcores=NS)

    @functools.partial(
        pl.kernel,
        out_type=jax.ShapeDtypeStruct((NC * NPAD,), jnp.float32),
        mesh=mesh,
        scratch_types=[
            [pltpu.VMEM((EC,), jnp.int32)] * 4,
            pltpu.VMEM((EC,), jnp.float32),
            pltpu.VMEM((RPT,), jnp.float32),
            pltpu.VMEM_SHARED((NPAD,), jnp.float32),
            [pltpu.SemaphoreType.DMA] * 4,
            [pltpu.SemaphoreType.DMA] * 4,
        ],
    )
    def _deg_sc(dst_hbm, out_hbm, idx, ones_v, zbuf_v, acc, si, ss):
        cid = lax.axis_index("c")
        sid = lax.axis_index("s")
        wid = cid * NS + sid

        def iload(i, b):
            base = pl.multiple_of(wid * EPW + i * EC, 8)
            pltpu.async_copy(dst_hbm.at[pl.ds(base, EC)], idx[b], si[b])

        def iwait(b):
            pltpu.make_async_copy(
                dst_hbm.at[pl.ds(pl.multiple_of(wid * EPW, 8), EC)],
                idx[b], si[b]).wait()

        def scat(i, b):
            pltpu.async_copy(ones_v, acc.at[idx[b]], ss[b], add=True)

        def swait(b):
            pltpu.make_async_copy(ones_v, acc.at[idx[b]], ss[b]).wait()

        for b in range(2):
            iload(b, b)
        for j in range(RPT // 16):
            zbuf_v[pl.ds(j * 16, 16)] = jnp.zeros((16,), jnp.float32)
        for j in range(EC // 16):
            ones_v[pl.ds(j * 16, 16)] = jnp.ones((16,), jnp.float32)
        pltpu.sync_copy(zbuf_v, acc.at[pl.ds(sid * RPT, RPT)])
        plsc.subcore_barrier()

        for i in (0, 1):
            iload(i + 2, (i + 2) % 4)
            iwait(i % 4)
            scat(i, i % 4)
        for i in (2, 3):
            c = (i + 2) % 4
            swait(c)
            iload(i + 2, c)
            iwait(i % 4)
            scat(i, i % 4)

        def body(g, carry):
            for b in range(4):
                i = 4 * g + b
                c = (b + 2) % 4
                swait(c)
                iload(i + 2, c)
                iwait(b)
                scat(i, b)
            return carry

        lax.fori_loop(1, 30, body, 0)
        for i in (120, 121, 122):
            b = i % 4
            c = (i + 2) % 4
            swait(c)
            iload(i + 2, c)
            iwait(b)
            scat(i, b)
        for i in (123, 124):
            b = i % 4
            iwait(b)
            scat(i, b)
        for b in (1, 2, 3, 0):
            swait(b)
        plsc.subcore_barrier()
        dst0 = pl.multiple_of(cid * NPAD + sid * RPT, 8)
        pltpu.sync_copy(acc.at[pl.ds(sid * RPT, RPT)],
                        out_hbm.at[pl.ds(dst0, RPT)])

    @functools.partial(
        pl.kernel,
        out_type=jax.ShapeDtypeStruct((NC * NPAD, D), jnp.float32),
        mesh=mesh,
        scratch_types=[
            [pltpu.VMEM((EC,), jnp.int32)] * 8,
            [pltpu.VMEM((EC,), jnp.int32)] * 8,
            [pltpu.VMEM((EC, D), jnp.float32)] * 4,
            pltpu.VMEM((32, D), jnp.float32),
            pltpu.VMEM_SHARED((NPAD, D), jnp.float32),
            [pltpu.SemaphoreType.DMA] * 8,
            [pltpu.SemaphoreType.DMA] * 4,
            [pltpu.SemaphoreType.DMA] * 4,
        ],
    )
    def _hop_sc(src_hbm, dst_hbm, y_hbm, out_hbm, sidx, didx, rows,
                zbuf_v, acc, si, sg, ss):
        cid = lax.axis_index("c")
        sid = lax.axis_index("s")
        wid = cid * NS + sid

        def iload(i, u):
            base = pl.multiple_of(wid * EPW + i * EC, 8)
            pltpu.async_copy(src_hbm.at[pl.ds(base, EC)], sidx[u], si[u])
            pltpu.async_copy(dst_hbm.at[pl.ds(base, EC)], didx[u], si[u])

        def iwait(u):
            base = pl.multiple_of(wid * EPW, 8)
            pltpu.make_async_copy(
                src_hbm.at[pl.ds(base, EC)], sidx[u], si[u]).wait()
            pltpu.make_async_copy(
                dst_hbm.at[pl.ds(base, EC)], didx[u], si[u]).wait()

        def gat(u, r):
            pltpu.async_copy(y_hbm.at[sidx[u]], rows[r], sg[r])

        def gwait(r):
            pltpu.make_async_copy(
                y_hbm.at[sidx[0]], rows[r], sg[r]).wait()

        def scat(u, r):
            pltpu.async_copy(rows[r], acc.at[didx[u]], ss[r], add=True)

        def swait(u, r):
            pltpu.make_async_copy(rows[r], acc.at[didx[u]], ss[r]).wait()

        for j in range(4):
            iload(j, j)

        def zfill(i, carry):
            for j in range(D // 16):
                zbuf_v[i, pl.ds(j * 16, 16)] = jnp.zeros((16,), jnp.float32)
            return carry

        lax.fori_loop(0, 32, zfill, 0)
        iwait(0)
        gat(0, 0)
        iwait(1)
        gat(1, 1)
        for i in range(RPT // 32):
            pltpu.sync_copy(zbuf_v, acc.at[pl.ds(sid * RPT + i * 32, 32)])
        plsc.subcore_barrier()

        def step(i, u, do_swait=True, do_iload=True, do_gat=True):
            if do_swait:
                swait((u + 2) % 8, (u + 2) % 4)
            if do_iload:
                iload(i + 4, (u + 4) % 8)
            if do_gat:
                iwait((u + 2) % 8)
                gat((u + 2) % 8, (u + 2) % 4)
            gwait(u % 4)
            scat(u, u % 4)

        for u in (0, 1):
            step(u, u, do_swait=False)
        for u in range(2, 8):
            step(u, u)

        def body(g, carry):
            for u in range(8):
                step(8 * g + u, u)
            return carry

        lax.fori_loop(1, 14, body, 0)
        for i in range(112, 121):
            step(i, i % 8)
        for i in (121, 122):
            step(i, i % 8, do_iload=False)
        for i in (123, 124):
            step(i, i % 8, do_iload=False, do_gat=False)
        swait(123 % 8, 3)
        swait(124 % 8, 0)
        plsc.subcore_barrier()
        dst0 = pl.multiple_of(cid * NPAD + sid * RPT, 8)
        pltpu.sync_copy(acc.at[pl.ds(sid * RPT, RPT)],
                        out_hbm.at[pl.ds(dst0, RPT)])

    @functools.partial(
        pl.kernel,
        out_type=(jax.ShapeDtypeStruct((TOT, D), jnp.float32),
                  jax.ShapeDtypeStruct((TOT, D), jnp.float32)),
        mesh=mesh,
        scratch_types=[
            [pltpu.VMEM((GC,), jnp.int32)] * 4,
            [pltpu.VMEM((GC, D), jnp.float32)] * 4,
            [pltpu.SemaphoreType.DMA] * 4,
            [pltpu.SemaphoreType.DMA] * 4,
        ],
    )
    def _pool_gather_sc(z_hbm, hidx_hbm, tidx_hbm, outh_hbm, outt_hbm,
                        gidx, grow, sg, ws):
        cid = lax.axis_index("c")
        sid = lax.axis_index("s")
        wid = cid * NS + sid

        def gwait(b):
            pltpu.make_async_copy(
                z_hbm.at[gidx[b]], grow[b], sg[b]).wait()

        def wwait(b):
            pltpu.make_async_copy(
                grow[b], outh_hbm.at[pl.ds(0, GC)], ws[b]).wait()

        for j in range(4):
            base = pl.multiple_of(wid * GPW + j * GC, 8)
            pltpu.sync_copy(hidx_hbm.at[pl.ds(base, GC)], gidx[j])
            pltpu.async_copy(z_hbm.at[gidx[j]], grow[j], sg[j])
        for j in range(4):
            base = pl.multiple_of(wid * GPW + j * GC, 8)
            gwait(j)
            pltpu.async_copy(grow[j], outh_hbm.at[pl.ds(base, GC)], ws[j])
        for j in range(4):
            base = pl.multiple_of(wid * GPW + j * GC, 8)
            wwait(j)
            pltpu.sync_copy(tidx_hbm.at[pl.ds(base, GC)], gidx[j])
            pltpu.async_copy(z_hbm.at[gidx[j]], grow[j], sg[j])
        for j in range(4):
            base = pl.multiple_of(wid * GPW + j * GC, 8)
            gwait(j)
            pltpu.async_copy(grow[j], outt_hbm.at[pl.ds(base, GC)], ws[j])
        for j in range(4):
            wwait(j)

    return _deg_sc, _hop_sc, _pool_gather_sc


def _pre_tc(p0_ref, p1_ref, emb_ref, b0_ref, dinv_ref, y_ref, h_ref):
    deg = p0_ref[...] + p1_ref[...] + 1.0
    dinv = lax.rsqrt(jnp.maximum(deg, 1.0))
    dinv_ref[...] = dinv
    y_ref[...] = emb_ref[...] * dinv
    h_ref[...] = emb_ref[...] * b0_ref[0, 0]


def _combine_tc(p0_ref, p1_ref, y_ref, h_ref, dinv_ref, bk_ref,
                hout_ref, yout_ref):
    dinv = dinv_ref[...]
    x = (p0_ref[...] + p1_ref[...] + y_ref[...]) * dinv
    hout_ref[...] = h_ref[...] + bk_ref[0, 0] * x
    yout_ref[...] = x * dinv


def _pool_mlp_tc(subh_ref, subt_ref, hseg_ref, tseg_ref, aw_ref, ab_ref,
                 w1_ref, b1_ref, w2_ref, b2_ref, out_ref):
    hp = lax.Precision.HIGHEST
    aw = aw_ref[...]
    ab = ab_ref[0, 0]

    def pool(sub, seg_row):
        s = lax.dot_general(aw, sub, (((0,), (1,)), ((), ())),
                            precision=hp,
                            preferred_element_type=jnp.float32) + ab
        onehot = seg_row == lax.broadcasted_iota(jnp.int32, (NSETS, TOT), 0)
        m = jnp.max(jnp.where(onehot, s, -3e38), axis=1, keepdims=True)
        m_safe = jnp.where(m > -1e37, m, 0.0)
        m_per = jnp.sum(jnp.where(onehot, m_safe, 0.0), axis=0,
                        keepdims=True)
        e = jnp.exp(s - m_per)
        denom = jnp.sum(jnp.where(onehot, e, 0.0), axis=1, keepdims=True)
        safe_denom = jnp.where(denom > 0, denom, 1.0)
        wden = jnp.sum(jnp.where(onehot, safe_denom, 0.0), axis=0,
                       keepdims=True)
        w = e / wden
        wm = jnp.where(onehot, w, 0.0)
        pooled = lax.dot_general(wm, sub, (((1,), (0,)), ((), ())),
                                 precision=hp,
                                 preferred_element_type=jnp.float32)
        cnt = jnp.sum(jnp.where(onehot, 1.0, 0.0), axis=1, keepdims=True)
        return jnp.where(cnt > 0, pooled, 0.0)

    h = pool(subh_ref[...], hseg_ref[...])
    t = pool(subt_ref[...], tseg_ref[...])
    feats = jnp.concatenate([h, t, h * t], axis=-1)
    hidden = jnp.dot(feats, w1_ref[...], precision=hp,
                     preferred_element_type=jnp.float32) + b1_ref[...]
    hidden = jnp.maximum(hidden, 0.0)
    out_ref[...] = jnp.dot(hidden, w2_ref[...], precision=hp,
                           preferred_element_type=jnp.float32) + b2_ref[...]


def _tc_call(body, out_shapes, interpret=False):
    return pl.pallas_call(body, out_shape=out_shapes, interpret=interpret)


def kernel(embed, edge_index, H_idx, H_segment_ids, T_idx, T_segment_ids,
           beta, attn_w, attn_b, W1, b1, W2, b2):
    f32 = jnp.float32
    src = edge_index[0]
    dst = edge_index[1]
    _deg_sc, _hop_sc, _pool_gather_sc = _sc_kernels()

    degp = _deg_sc(dst)
    p0 = degp[:N].reshape(N, 1)
    p1 = degp[NPAD:NPAD + N].reshape(N, 1)

    dinv, y, h = _tc_call(
        _pre_tc,
        (jax.ShapeDtypeStruct((N, 1), f32),
         jax.ShapeDtypeStruct((N, D), f32),
         jax.ShapeDtypeStruct((N, D), f32)),
    )(p0, p1, embed, beta[0].reshape(1, 1))

    for k in range(KHOPS):
        parts = _hop_sc(src, dst, y)
        h, y = _tc_call(
            _combine_tc,
            (jax.ShapeDtypeStruct((N, D), f32),
             jax.ShapeDtypeStruct((N, D), f32)),
        )(parts[:N], parts[NPAD:NPAD + N], y, h, dinv,
          beta[k + 1].reshape(1, 1))

    subh, subt = _pool_gather_sc(h, H_idx, T_idx)

    logits = _tc_call(
        _pool_mlp_tc, jax.ShapeDtypeStruct((NSETS, NREL), f32),
    )(subh, subt,
      H_segment_ids.reshape(1, TOT), T_segment_ids.reshape(1, TOT),
      attn_w, attn_b.reshape(1, 1),
      W1, b1.reshape(1, HID), W2, b2.reshape(1, NREL))
    return logits
